# Initial kernel scaffold; baseline (speedup 1.0000x reference)
import jax
import jax.numpy as jnp
from jax.experimental import pallas as pl


def kernel(input_idx, target, embedding_table):
    raise NotImplementedError("write your pallas kernel here")


# speedup vs baseline: 1.3622x; 1.3622x over previous
import functools

import jax
import jax.numpy as jnp
from jax import lax
from jax.experimental import pallas as pl
from jax.experimental.pallas import tpu as pltpu
from jax.experimental.pallas import tpu_sc as plsc

V = 1000
D = 1000
DP = 1024
NW = 32
CH = 32
RB = 400


def _sc_body(table_p, fidx, out, idx_v, rows_v, sem):
    wid = lax.axis_index("s") * 2 + lax.axis_index("c")
    nrows = out.shape[0] // NW
    base = wid * nrows

    def chunk(g, carry):
        off = base + g * CH
        pltpu.sync_copy(fidx.at[pl.ds(off, CH)], idx_v)
        pltpu.async_copy(table_p.at[idx_v], rows_v, sem).wait()
        pltpu.sync_copy(rows_v, out.at[pl.ds(off, CH)])
        return carry

    lax.fori_loop(0, nrows // CH, chunk, jnp.int32(0))


def _finish_body(in_ref, tgt_ref, out_ref, part_ref):
    x = in_ref[:, pl.ds(0, D)]
    out_ref[...] = x
    m = jnp.max(x, axis=1, keepdims=True)
    s = jnp.sum(jnp.exp(x - m), axis=1, keepdims=True)
    lse = m + jnp.log(s)
    cols = lax.broadcasted_iota(jnp.int32, (RB, D), 1)
    tl = jnp.sum(jnp.where(cols == tgt_ref[...], x, 0.0), axis=1,
                 keepdims=True)
    part_ref[...] = jnp.reshape(jnp.sum(lse - tl), (1, 1, 1))


def kernel(input_idx, target, embedding_table):
    B, T = input_idx.shape
    N = B * T
    fidx = input_idx.reshape(N)
    table_p = jnp.pad(embedding_table, ((0, 0), (0, DP - D)))

    mesh = plsc.VectorSubcoreMesh(core_axis_name="c", subcore_axis_name="s")
    sc = pl.kernel(
        _sc_body,
        out_type=jax.ShapeDtypeStruct((N, DP), jnp.float32),
        mesh=mesh,
        scratch_types=[
            pltpu.VMEM((CH,), jnp.int32),
            pltpu.VMEM((CH, DP), jnp.float32),
            pltpu.SemaphoreType.DMA,
        ],
    )
    padded = sc(table_p, fidx)

    nb = N // RB
    logits_flat, part = pl.pallas_call(
        _finish_body,
        grid=(nb,),
        in_specs=[
            pl.BlockSpec((RB, DP), lambda i: (i, 0)),
            pl.BlockSpec((RB, 1), lambda i: (i, 0)),
        ],
        out_specs=[
            pl.BlockSpec((RB, D), lambda i: (i, 0)),
            pl.BlockSpec((1, 1, 1), lambda i: (i, 0, 0)),
        ],
        out_shape=[
            jax.ShapeDtypeStruct((N, D), jnp.float32),
            jax.ShapeDtypeStruct((nb, 1, 1), jnp.float32),
        ],
    )(padded, target.reshape(N, 1))
    loss = jnp.sum(part) / jnp.float32(N)
    return logits_flat.reshape(B, T, D), loss


# speedup vs baseline: 1.5201x; 1.1159x over previous
import functools

import jax
import jax.numpy as jnp
from jax import lax
from jax.experimental import pallas as pl
from jax.experimental.pallas import tpu as pltpu
from jax.experimental.pallas import tpu_sc as plsc

V = 1000
D = 1000
DP = 1024
NW = 32
CH = 32
RB = 400
T_LEN = 50


def _sc_body(table_p, fidx, out, idx_v, rows_v, sem):
    wid = lax.axis_index("s") * 2 + lax.axis_index("c")
    nrows = out.shape[0] // NW
    base = wid * nrows

    def chunk(g, carry):
        off = base + g * CH
        pltpu.sync_copy(fidx.at[pl.ds(off, CH)], idx_v)
        pltpu.async_copy(table_p.at[idx_v], rows_v, sem).wait()
        pltpu.sync_copy(rows_v, out.at[pl.ds(off, CH)])
        return carry

    lax.fori_loop(0, nrows // CH, chunk, jnp.int32(0))


def _finish_body(in_ref, tgt_ref, out_ref, part_ref):
    x = in_ref[:, pl.ds(0, D)]
    out_ref[...] = x.reshape(RB // T_LEN, T_LEN, D)
    m = jnp.max(x, axis=1, keepdims=True)
    s = jnp.sum(jnp.exp(x - m), axis=1, keepdims=True)
    lse = m + jnp.log(s)
    cols = lax.broadcasted_iota(jnp.int32, (RB, D), 1)
    tl = jnp.sum(jnp.where(cols == tgt_ref[...], x, 0.0), axis=1,
                 keepdims=True)
    part_ref[...] = jnp.reshape(jnp.sum(lse - tl), (1, 1, 1))


def kernel(input_idx, target, embedding_table):
    B, T = input_idx.shape
    N = B * T
    fidx = input_idx.reshape(N)
    table_p = jnp.pad(embedding_table, ((0, 0), (0, DP - D)))

    mesh = plsc.VectorSubcoreMesh(core_axis_name="c", subcore_axis_name="s")
    sc = pl.kernel(
        _sc_body,
        out_type=jax.ShapeDtypeStruct((N, DP), jnp.float32),
        mesh=mesh,
        scratch_types=[
            pltpu.VMEM((CH,), jnp.int32),
            pltpu.VMEM((CH, DP), jnp.float32),
            pltpu.SemaphoreType.DMA,
        ],
    )
    padded = sc(table_p, fidx)

    nb = N // RB
    logits_flat, part = pl.pallas_call(
        _finish_body,
        grid=(nb,),
        in_specs=[
            pl.BlockSpec((RB, DP), lambda i: (i, 0)),
            pl.BlockSpec((RB, 1), lambda i: (i, 0)),
        ],
        out_specs=[
            pl.BlockSpec((RB // T_LEN, T_LEN, D), lambda i: (i, 0, 0)),
            pl.BlockSpec((1, 1, 1), lambda i: (i, 0, 0)),
        ],
        out_shape=[
            jax.ShapeDtypeStruct((B, T, D), jnp.float32),
            jax.ShapeDtypeStruct((nb, 1, 1), jnp.float32),
        ],
    )(padded, target.reshape(N, 1))
    loss = jnp.sum(part) / jnp.float32(N)
    return logits_flat, loss
